# baseline (speedup 1.0000x reference)
import jax
import jax.numpy as jnp
from jax import lax
from jax.experimental import pallas as pl

N = 10000
D = 128

_RB = 1000
_GRID = N // _RB


def _tc_a_body(x_ref, w_ref, deg_ref, o_ref, dis_ref):
    deg = deg_ref[...] + 1.0
    dis = lax.rsqrt(deg)
    h = jnp.dot(x_ref[...], w_ref[...], preferred_element_type=jnp.float32)
    o_ref[...] = h * dis
    dis_ref[...] = dis


def _tc_b_body(acc_ref, g_ref, dis_ref, b_ref, w_ref, o_ref):
    dis = dis_ref[...]
    t = (acc_ref[...] + g_ref[...]) * dis + b_ref[...]
    h2 = jnp.maximum(t, 0.0)
    o_ref[...] = jnp.dot(h2, w_ref[...],
                         preferred_element_type=jnp.float32) * dis


def _tc_c_body(acc_ref, g_ref, dis_ref, b_ref, m_ref, o_ref):
    t = (acc_ref[...] + g_ref[...]) * dis_ref[...] + b_ref[...]
    o_ref[...] = t * m_ref[...]


_spec_rows = pl.BlockSpec((_RB, D), lambda i: (i, 0))
_spec_w = pl.BlockSpec((D, D), lambda i: (0, 0))
_spec_col = pl.BlockSpec((_RB, 1), lambda i: (i, 0))
_spec_b = pl.BlockSpec((1, D), lambda i: (0, 0))
_out_rows = jax.ShapeDtypeStruct((N, D), jnp.float32)
_out_col = jax.ShapeDtypeStruct((N, 1), jnp.float32)

_tc_a = pl.pallas_call(
    _tc_a_body, grid=(_GRID,),
    in_specs=[_spec_rows, _spec_w, _spec_col],
    out_specs=[_spec_rows, _spec_col], out_shape=[_out_rows, _out_col])

_tc_b = pl.pallas_call(
    _tc_b_body, grid=(_GRID,),
    in_specs=[_spec_rows, _spec_rows, _spec_col, _spec_b, _spec_w],
    out_specs=_spec_rows, out_shape=_out_rows)

_tc_c = pl.pallas_call(
    _tc_c_body, grid=(_GRID,),
    in_specs=[_spec_rows, _spec_rows, _spec_col, _spec_b, _spec_col],
    out_specs=_spec_rows, out_shape=_out_rows)


def kernel(x, edge_index, boundary_mask, W1, b1, W2, b2):
    src = edge_index[0].astype(jnp.int32)
    dst = edge_index[1].astype(jnp.int32)
    dsts, srcs = jax.lax.sort([dst, src], num_keys=1)

    bounds = jnp.searchsorted(dsts, jnp.arange(N + 1, dtype=jnp.int32))
    deg = jnp.diff(bounds).astype(jnp.float32).reshape(N, 1)
    maskf = jnp.where(boundary_mask, 0.0, 1.0).astype(jnp.float32)
    maskf = maskf.reshape(N, 1)

    def _edge_pass(g):
        msg = jnp.take(g, srcs, axis=0, mode="clip")
        return jnp.zeros((N, D), jnp.float32).at[dsts].add(
            msg, indices_are_sorted=True, mode="promise_in_bounds")

    g1, dis = _tc_a(x, W1, deg)
    acc1 = _edge_pass(g1)
    g2 = _tc_b(acc1, g1, dis, b1.reshape(1, D), W2)
    acc2 = _edge_pass(g2)
    out = _tc_c(acc2, g2, dis, b2.reshape(1, D), maskf)
    return out


# speedup vs baseline: 1.2867x; 1.2867x over previous
import jax
import jax.numpy as jnp
from jax import lax
from jax.experimental import pallas as pl

N = 10000
D = 128

_RB = 1000
_GRID = N // _RB


def _tc_a_body(x_ref, w_ref, deg_ref, o_ref, dis_ref):
    deg = deg_ref[...] + 1.0
    dis = lax.rsqrt(deg)
    h = jnp.dot(x_ref[...], w_ref[...], preferred_element_type=jnp.float32)
    o_ref[...] = h * dis
    dis_ref[...] = dis


def _tc_b_body(acc_ref, g_ref, dis_ref, b_ref, w_ref, o_ref):
    dis = dis_ref[...]
    t = (acc_ref[...] + g_ref[...]) * dis + b_ref[...]
    h2 = jnp.maximum(t, 0.0)
    o_ref[...] = jnp.dot(h2, w_ref[...],
                         preferred_element_type=jnp.float32) * dis


def _tc_c_body(acc_ref, g_ref, dis_ref, b_ref, m_ref, o_ref):
    t = (acc_ref[...] + g_ref[...]) * dis_ref[...] + b_ref[...]
    o_ref[...] = t * m_ref[...]


_spec_rows = pl.BlockSpec((_RB, D), lambda i: (i, 0))
_spec_w = pl.BlockSpec((D, D), lambda i: (0, 0))
_spec_col = pl.BlockSpec((_RB, 1), lambda i: (i, 0))
_spec_b = pl.BlockSpec((1, D), lambda i: (0, 0))
_out_rows = jax.ShapeDtypeStruct((N, D), jnp.float32)
_out_col = jax.ShapeDtypeStruct((N, 1), jnp.float32)

_tc_a = pl.pallas_call(
    _tc_a_body, grid=(_GRID,),
    in_specs=[_spec_rows, _spec_w, _spec_col],
    out_specs=[_spec_rows, _spec_col], out_shape=[_out_rows, _out_col])

_tc_b = pl.pallas_call(
    _tc_b_body, grid=(_GRID,),
    in_specs=[_spec_rows, _spec_rows, _spec_col, _spec_b, _spec_w],
    out_specs=_spec_rows, out_shape=_out_rows)

_tc_c = pl.pallas_call(
    _tc_c_body, grid=(_GRID,),
    in_specs=[_spec_rows, _spec_rows, _spec_col, _spec_b, _spec_col],
    out_specs=_spec_rows, out_shape=_out_rows)


def kernel(x, edge_index, boundary_mask, W1, b1, W2, b2):
    src = edge_index[0].astype(jnp.int32)
    dst = edge_index[1].astype(jnp.int32)
    dsts, srcs = jax.lax.sort([dst, src], num_keys=1)

    deg = jnp.zeros((N, 1), jnp.float32).at[dsts].add(
        1.0, indices_are_sorted=True, mode="promise_in_bounds")
    maskf = jnp.where(boundary_mask, 0.0, 1.0).astype(jnp.float32)
    maskf = maskf.reshape(N, 1)

    def _edge_pass(g):
        msg = jnp.take(g, srcs, axis=0, mode="clip")
        return jnp.zeros((N, D), jnp.float32).at[dsts].add(
            msg, indices_are_sorted=True, mode="promise_in_bounds")

    g1, dis = _tc_a(x, W1, deg)
    acc1 = _edge_pass(g1)
    g2 = _tc_b(acc1, g1, dis, b1.reshape(1, D), W2)
    acc2 = _edge_pass(g2)
    out = _tc_c(acc2, g2, dis, b2.reshape(1, D), maskf)
    return out
